# baseline (speedup 1.0000x reference)
import functools

import jax
import jax.numpy as jnp
from jax import lax
from jax.experimental import pallas as pl
from jax.experimental.pallas import tpu as pltpu
from jax.experimental.pallas import tpu_sc as plsc

NUM_EMB = 1000000
NUM_TOKENS = 16384
EMBED_DIM = 32
NUM_TAGS = 32

_info = plsc.get_sparse_core_info()
_NC, _NS = _info.num_cores, _info.num_subcores
_NW = _NC * _NS
_BPW = NUM_TOKENS // _NW

_TN = 8192


def _ts_body(w_ref, b_ref, tT_ref, o_ref):
  o_ref[...] = (
      lax.dot_general(tT_ref[...], w_ref[...], (((0,), (1,)), ((), ())),
                      preferred_element_type=jnp.float32)
      + b_ref[...])


def _score_table(W, b2d, tableT):
  return pl.pallas_call(
      _ts_body,
      grid=(pl.cdiv(NUM_EMB, _TN),),
      in_specs=[
          pl.BlockSpec((NUM_TAGS, EMBED_DIM), lambda i: (0, 0)),
          pl.BlockSpec((1, NUM_TAGS), lambda i: (0, 0)),
          pl.BlockSpec((EMBED_DIM, _TN), lambda i: (0, i)),
      ],
      out_specs=pl.BlockSpec((_TN, NUM_TAGS), lambda i: (i, 0)),
      out_shape=jax.ShapeDtypeStruct((NUM_EMB, NUM_TAGS), jnp.float32),
  )(W, b2d, tableT)


def _make_gather():
  mesh = plsc.VectorSubcoreMesh(core_axis_name="c", subcore_axis_name="s")

  @functools.partial(
      pl.kernel,
      mesh=mesh,
      out_type=jax.ShapeDtypeStruct((NUM_TOKENS, EMBED_DIM), jnp.float32),
      scratch_types=[
          pltpu.VMEM((_BPW,), jnp.int32),
          pltpu.VMEM((_BPW, EMBED_DIM), jnp.float32),
          pltpu.SemaphoreType.DMA,
      ],
  )
  def gather_k(idx_hbm, table_hbm, out_hbm, idx_v, rows_v, sem):
    wid = lax.axis_index("s") * _NC + lax.axis_index("c")
    base = wid * _BPW
    pltpu.sync_copy(idx_hbm.at[pl.ds(base, _BPW)], idx_v)

    def body(g, carry):
      vec = idx_v[pl.ds(g * 16, 16)]
      for j in range(16):
        r = lax.squeeze(lax.slice(vec, (j,), (j + 1,)), (0,))
        pltpu.async_copy(
            table_hbm.at[pl.ds(r, 1)], rows_v.at[pl.ds(g * 16 + j, 1)], sem)
      return carry

    lax.fori_loop(0, _BPW // 16, body, 0)
    pltpu.make_async_copy(table_hbm.at[pl.ds(0, _BPW)], rows_v, sem).wait()
    pltpu.sync_copy(rows_v, out_hbm.at[pl.ds(base, _BPW)])

  return gather_k


_gather = _make_gather()


def kernel(sent, emb_table, W, b):
  tableT = jnp.swapaxes(emb_table, 0, 1)
  scores_all = _score_table(W, b.reshape(1, NUM_TAGS), tableT)
  return _gather(sent, scores_all)
